# baseline (speedup 1.0000x reference)
import functools

import jax
import jax.numpy as jnp
from jax import lax
from jax.experimental import pallas as pl
from jax.experimental.pallas import tpu as pltpu
from jax.experimental.pallas import tpu_sc as plsc

B = 4096
L = 200
D = 64
HID = 128
NUM_CLASSES = 2

NC = 2
NS = 16
NW = NC * NS
B_PER_W = B // NW
CHUNK = 100
CHUNKS_PER_W = B_PER_W * 2
ROW_UNROLL = 4

_mesh = plsc.VectorSubcoreMesh(core_axis_name="c", subcore_axis_name="s")


@functools.partial(
    pl.kernel,
    out_type=jax.ShapeDtypeStruct((B, D), jnp.float32),
    mesh=_mesh,
    compiler_params=pltpu.CompilerParams(use_tc_tiling_on_sc=False),
    scratch_types=[
        pltpu.VMEM((CHUNKS_PER_W, CHUNK), jnp.int32),
        pltpu.VMEM((CHUNK, D), jnp.float32),
        pltpu.VMEM((CHUNK, D), jnp.float32),
        pltpu.VMEM((B_PER_W, D), jnp.float32),
        pltpu.SemaphoreType.DMA,
        pltpu.SemaphoreType.DMA,
    ],
)
def _pool(x_hbm, emb_hbm, out_hbm, idx_v, buf0, buf1, m_v, sem0, sem1):
    wid = lax.axis_index("s") * NC + lax.axis_index("c")
    cbase = wid * CHUNKS_PER_W

    pltpu.sync_copy(x_hbm.at[pl.ds(cbase, CHUNKS_PER_W)], idx_v)

    bufs = (buf0, buf1)
    sems = (sem0, sem1)

    def start(chunk_i, half):
        pltpu.make_async_copy(
            emb_hbm.at[idx_v.at[chunk_i]], bufs[half], sems[half]
        ).start()

    def wait(chunk_i, half):
        pltpu.make_async_copy(
            emb_hbm.at[idx_v.at[chunk_i]], bufs[half], sems[half]
        ).wait()

    def accum(buf, acc):
        def row_body(r, a):
            a0, a1, a2, a3 = a
            for u in range(ROW_UNROLL):
                row = r * ROW_UNROLL + u
                a0 = a0 + buf[row, pl.ds(0, 16)]
                a1 = a1 + buf[row, pl.ds(16, 16)]
                a2 = a2 + buf[row, pl.ds(32, 16)]
                a3 = a3 + buf[row, pl.ds(48, 16)]
            return (a0, a1, a2, a3)

        return lax.fori_loop(0, CHUNK // ROW_UNROLL, row_body, acc)

    start(0, 0)
    start(1, 1)

    inv_l = jnp.float32(1.0 / L)
    zero = jnp.zeros((16,), jnp.float32)

    def sample_body(i, _):
        c0 = i * 2
        wait(c0, 0)
        acc = accum(buf0, (zero, zero, zero, zero))

        @pl.when(i < B_PER_W - 1)
        def _():
            start(c0 + 2, 0)

        wait(c0 + 1, 1)
        acc = accum(buf1, acc)

        @pl.when(i < B_PER_W - 1)
        def _():
            start(c0 + 3, 1)

        m_v[i, pl.ds(0, 16)] = acc[0] * inv_l
        m_v[i, pl.ds(16, 16)] = acc[1] * inv_l
        m_v[i, pl.ds(32, 16)] = acc[2] * inv_l
        m_v[i, pl.ds(48, 16)] = acc[3] * inv_l
        return 0

    lax.fori_loop(0, B_PER_W, sample_body, 0)

    pltpu.sync_copy(m_v, out_hbm.at[pl.ds(wid * B_PER_W, B_PER_W)])


def _mlp_body(m_ref, vw_ref, vb_ref, ww_ref, wb_ref, out_ref):
    m = m_ref[...]
    h = jnp.dot(m, vw_ref[...], preferred_element_type=jnp.float32)
    h = jnp.maximum(h + vb_ref[...], 0.0)
    logits = jnp.dot(h, ww_ref[...], preferred_element_type=jnp.float32)
    logits = logits + wb_ref[...]
    mx = jnp.max(logits, axis=1, keepdims=True)
    s = logits - mx
    lse = jnp.log(jnp.sum(jnp.exp(s), axis=1, keepdims=True))
    out_ref[...] = s - lse


def _mlp(m, vw_t, vb, ww_t, wb):
    return pl.pallas_call(
        _mlp_body,
        out_shape=jax.ShapeDtypeStruct((B, NUM_CLASSES), jnp.float32),
    )(m, vw_t, vb, ww_t, wb)


@jax.jit
def kernel(x, emb, V_w, V_b, W_w, W_b):
    x2 = x.astype(jnp.int32).reshape(B * 2, CHUNK)
    m = _pool(x2, emb)
    return _mlp(m, V_w.T, V_b.reshape(1, HID), W_w.T, W_b.reshape(1, NUM_CLASSES))
